# Initial kernel scaffold; baseline (speedup 1.0000x reference)
import jax
import jax.numpy as jnp
from jax.experimental import pallas as pl


def kernel(x, edge_index, W1z, b1z, W1r, b1r, W1h, b1h, W2z, b2z, W2r, b2r, W2h, b2h):
    raise NotImplementedError("write your pallas kernel here")


# speedup vs baseline: 2.0457x; 2.0457x over previous
import jax
import jax.numpy as jnp
from jax.experimental import pallas as pl
from jax.experimental.pallas import tpu as pltpu

N = 10000
IN = 256
OUT = 128
H1 = 256
TN = 400


def _fused_encoder_kernel(x_ref, wc1_ref, bc1_ref, wc2_ref, bc2_ref, out_ref):
    x = x_ref[...]
    p = jnp.dot(x, wc1_ref[...], preferred_element_type=jnp.float32) + bc1_ref[...]
    z1 = jax.nn.sigmoid(p[:, :H1])
    t1 = jnp.tanh(p[:, H1:])
    h = jax.nn.relu((1.0 - z1) * t1)
    q = jnp.dot(h, wc2_ref[...], preferred_element_type=jnp.float32) + bc2_ref[...]
    out_ref[...] = (1.0 - jax.nn.sigmoid(q[:, :OUT])) * jnp.tanh(q[:, OUT:])


def kernel(x, edge_index, W1z, b1z, W1r, b1r, W1h, b1h, W2z, b2z, W2r, b2r, W2h, b2h):
    wc1 = jnp.concatenate(
        [(W1z[0, 0] + W1z[1, 0])[:IN], (W1h[0, 0] + W1h[1, 0])[:IN]], axis=1
    )
    bc1 = jnp.concatenate([b1z, b1h])[None, :]
    wc2 = jnp.concatenate(
        [(W2z[0, 0] + W2z[1, 0])[:H1], (W2h[0, 0] + W2h[1, 0])[:H1]], axis=1
    )
    bc2 = jnp.concatenate([b2z, b2h])[None, :]

    return pl.pallas_call(
        _fused_encoder_kernel,
        grid=(N // TN,),
        in_specs=[
            pl.BlockSpec((TN, IN), lambda i: (i, 0)),
            pl.BlockSpec((IN, 2 * H1), lambda i: (0, 0)),
            pl.BlockSpec((1, 2 * H1), lambda i: (0, 0)),
            pl.BlockSpec((H1, 2 * OUT), lambda i: (0, 0)),
            pl.BlockSpec((1, 2 * OUT), lambda i: (0, 0)),
        ],
        out_specs=pl.BlockSpec((TN, OUT), lambda i: (i, 0)),
        out_shape=jax.ShapeDtypeStruct((N, OUT), jnp.float32),
        compiler_params=pltpu.CompilerParams(
            dimension_semantics=("arbitrary",),
        ),
    )(x, wc1, bc1, wc2, bc2)


# speedup vs baseline: 2.8466x; 1.3915x over previous
import jax
import jax.numpy as jnp
from jax.experimental import pallas as pl
from jax.experimental.pallas import tpu as pltpu

N = 10000
IN = 256
OUT = 128
H1 = 256
TN = 1000


def _fused_encoder_kernel(x_ref, wc1_ref, bc1_ref, wc2_ref, bc2_ref, out_ref):
    x = x_ref[...]
    p = jnp.dot(x, wc1_ref[...], preferred_element_type=jnp.float32) + bc1_ref[...]
    z1 = jax.nn.sigmoid(p[:, :H1])
    t1 = jnp.tanh(p[:, H1:])
    h = jax.nn.relu((1.0 - z1) * t1)
    q = jnp.dot(h, wc2_ref[...], preferred_element_type=jnp.float32) + bc2_ref[...]
    out_ref[...] = (1.0 - jax.nn.sigmoid(q[:, :OUT])) * jnp.tanh(q[:, OUT:])


def kernel(x, edge_index, W1z, b1z, W1r, b1r, W1h, b1h, W2z, b2z, W2r, b2r, W2h, b2h):
    wc1 = jnp.concatenate(
        [(W1z[0, 0] + W1z[1, 0])[:IN], (W1h[0, 0] + W1h[1, 0])[:IN]], axis=1
    )
    bc1 = jnp.concatenate([b1z, b1h])[None, :]
    wc2 = jnp.concatenate(
        [(W2z[0, 0] + W2z[1, 0])[:H1], (W2h[0, 0] + W2h[1, 0])[:H1]], axis=1
    )
    bc2 = jnp.concatenate([b2z, b2h])[None, :]

    return pl.pallas_call(
        _fused_encoder_kernel,
        grid=(N // TN,),
        in_specs=[
            pl.BlockSpec((TN, IN), lambda i: (i, 0)),
            pl.BlockSpec((IN, 2 * H1), lambda i: (0, 0)),
            pl.BlockSpec((1, 2 * H1), lambda i: (0, 0)),
            pl.BlockSpec((H1, 2 * OUT), lambda i: (0, 0)),
            pl.BlockSpec((1, 2 * OUT), lambda i: (0, 0)),
        ],
        out_specs=pl.BlockSpec((TN, OUT), lambda i: (i, 0)),
        out_shape=jax.ShapeDtypeStruct((N, OUT), jnp.float32),
        compiler_params=pltpu.CompilerParams(
            dimension_semantics=("arbitrary",),
        ),
    )(x, wc1, bc1, wc2, bc2)


# speedup vs baseline: 3.1570x; 1.1090x over previous
import jax
import jax.numpy as jnp
from jax.experimental import pallas as pl
from jax.experimental.pallas import tpu as pltpu

N = 10000
IN = 256
OUT = 128
H1 = 256
TN = 2000


def _fused_encoder_kernel(x_ref, wc1_ref, bc1_ref, wc2_ref, bc2_ref, out_ref):
    x = x_ref[...]
    p = jnp.dot(x, wc1_ref[...], preferred_element_type=jnp.float32) + bc1_ref[...]
    z1 = jax.nn.sigmoid(p[:, :H1])
    t1 = jnp.tanh(p[:, H1:])
    h = jax.nn.relu((1.0 - z1) * t1)
    q = jnp.dot(h, wc2_ref[...], preferred_element_type=jnp.float32) + bc2_ref[...]
    out_ref[...] = (1.0 - jax.nn.sigmoid(q[:, :OUT])) * jnp.tanh(q[:, OUT:])


def kernel(x, edge_index, W1z, b1z, W1r, b1r, W1h, b1h, W2z, b2z, W2r, b2r, W2h, b2h):
    wc1 = jnp.concatenate(
        [(W1z[0, 0] + W1z[1, 0])[:IN], (W1h[0, 0] + W1h[1, 0])[:IN]], axis=1
    )
    bc1 = jnp.concatenate([b1z, b1h])[None, :]
    wc2 = jnp.concatenate(
        [(W2z[0, 0] + W2z[1, 0])[:H1], (W2h[0, 0] + W2h[1, 0])[:H1]], axis=1
    )
    bc2 = jnp.concatenate([b2z, b2h])[None, :]

    return pl.pallas_call(
        _fused_encoder_kernel,
        grid=(N // TN,),
        in_specs=[
            pl.BlockSpec((TN, IN), lambda i: (i, 0)),
            pl.BlockSpec((IN, 2 * H1), lambda i: (0, 0)),
            pl.BlockSpec((1, 2 * H1), lambda i: (0, 0)),
            pl.BlockSpec((H1, 2 * OUT), lambda i: (0, 0)),
            pl.BlockSpec((1, 2 * OUT), lambda i: (0, 0)),
        ],
        out_specs=pl.BlockSpec((TN, OUT), lambda i: (i, 0)),
        out_shape=jax.ShapeDtypeStruct((N, OUT), jnp.float32),
        compiler_params=pltpu.CompilerParams(
            dimension_semantics=("arbitrary",),
        ),
    )(x, wc1, bc1, wc2, bc2)


# speedup vs baseline: 3.2186x; 1.0195x over previous
import jax
import jax.numpy as jnp
from jax.experimental import pallas as pl
from jax.experimental.pallas import tpu as pltpu

N = 10000
IN = 256
OUT = 128
H1 = 256
TN = 5000


def _fused_encoder_kernel(x_ref, wc1_ref, bc1_ref, wc2_ref, bc2_ref, out_ref):
    x = x_ref[...]
    p = jnp.dot(x, wc1_ref[...], preferred_element_type=jnp.float32) + bc1_ref[...]
    z1 = jax.nn.sigmoid(p[:, :H1])
    t1 = jnp.tanh(p[:, H1:])
    h = jax.nn.relu((1.0 - z1) * t1)
    q = jnp.dot(h, wc2_ref[...], preferred_element_type=jnp.float32) + bc2_ref[...]
    out_ref[...] = (1.0 - jax.nn.sigmoid(q[:, :OUT])) * jnp.tanh(q[:, OUT:])


def kernel(x, edge_index, W1z, b1z, W1r, b1r, W1h, b1h, W2z, b2z, W2r, b2r, W2h, b2h):
    wc1 = jnp.concatenate(
        [(W1z[0, 0] + W1z[1, 0])[:IN], (W1h[0, 0] + W1h[1, 0])[:IN]], axis=1
    )
    bc1 = jnp.concatenate([b1z, b1h])[None, :]
    wc2 = jnp.concatenate(
        [(W2z[0, 0] + W2z[1, 0])[:H1], (W2h[0, 0] + W2h[1, 0])[:H1]], axis=1
    )
    bc2 = jnp.concatenate([b2z, b2h])[None, :]

    return pl.pallas_call(
        _fused_encoder_kernel,
        grid=(N // TN,),
        in_specs=[
            pl.BlockSpec((TN, IN), lambda i: (i, 0)),
            pl.BlockSpec((IN, 2 * H1), lambda i: (0, 0)),
            pl.BlockSpec((1, 2 * H1), lambda i: (0, 0)),
            pl.BlockSpec((H1, 2 * OUT), lambda i: (0, 0)),
            pl.BlockSpec((1, 2 * OUT), lambda i: (0, 0)),
        ],
        out_specs=pl.BlockSpec((TN, OUT), lambda i: (i, 0)),
        out_shape=jax.ShapeDtypeStruct((N, OUT), jnp.float32),
        compiler_params=pltpu.CompilerParams(
            dimension_semantics=("arbitrary",),
        ),
    )(x, wc1, bc1, wc2, bc2)
